# Initial kernel scaffold; baseline (speedup 1.0000x reference)
import jax
import jax.numpy as jnp
from jax.experimental import pallas as pl


def kernel(x_in, edge_index_in, ego_edge_index, W_ego1, b_ego1, W_gcn1, b_gcn1, W_ego2, b_ego2, W_gcn2, b_gcn2):
    raise NotImplementedError("write your pallas kernel here")


# speedup vs baseline: 5.2144x; 5.2144x over previous
import functools

import jax
import jax.numpy as jnp
from jax import lax
from jax.experimental import pallas as pl
from jax.experimental.pallas import tpu as pltpu
from jax.experimental.pallas import tpu_sc as plsc

_N = 10000
_NPAD = 10240
_NW = 32
_K = 128
_CH = 80
_EPAD = _NW * _K * _CH
_RPW = _NPAD // 16

_sc_mesh = plsc.VectorSubcoreMesh(core_axis_name="c", subcore_axis_name="s")


def _zero_fill(buf, nrows, ncols):
    zero = jnp.zeros((16,), jnp.float32)

    @pl.loop(0, nrows)
    def _(i):
        @pl.loop(0, ncols, step=16)
        def _(j):
            buf[i, pl.ds(j, 16)] = zero


@functools.partial(jax.jit, static_argnums=(3,))
def _segsum_sc(src_idx, dst_idx, table, F):
    @functools.partial(
        pl.kernel,
        out_type=jax.ShapeDtypeStruct((2, _NPAD, F), jnp.float32),
        mesh=_sc_mesh,
        compiler_params=pltpu.CompilerParams(use_tc_tiling_on_sc=False),
        scratch_types=[
            pltpu.VMEM((2, _K), jnp.int32),
            pltpu.VMEM((2, _K), jnp.int32),
            pltpu.VMEM((2, _K, F), jnp.float32),
            pltpu.VMEM((32, F), jnp.float32),
            pltpu.VMEM_SHARED((_NPAD, F), jnp.float32),
        ],
    )
    def k(src_hbm, dst_hbm, tab_hbm, out_hbm, sidx, didx, rows, zbuf, acc):
        c = lax.axis_index("c")
        s = lax.axis_index("s")
        wid = s * 2 + c

        _zero_fill(zbuf, 32, F)

        @pl.loop(0, _RPW, step=32)
        def _(r):
            pltpu.sync_copy(zbuf, acc.at[pl.ds(s * _RPW + r, 32)])

        plsc.subcore_barrier()

        @pl.loop(0, _CH)
        def _(j):
            pltpu.sync_copy(src_hbm.at[wid, j], sidx.at[0])
            pltpu.sync_copy(dst_hbm.at[wid, j], didx.at[0])
            pltpu.sync_copy(tab_hbm.at[sidx.at[0]], rows.at[0])
            pltpu.sync_copy(rows.at[0], acc.at[didx.at[0]], add=True)

        plsc.subcore_barrier()
        pltpu.sync_copy(acc.at[pl.ds(s * _RPW, _RPW)],
                        out_hbm.at[c].at[pl.ds(s * _RPW, _RPW)])

    return k(src_idx, dst_idx, table)


@jax.jit
def _hist_sc(dst_idx):
    @functools.partial(
        pl.kernel,
        out_type=jax.ShapeDtypeStruct((2, _NPAD, 16), jnp.float32),
        mesh=_sc_mesh,
        scratch_types=[
            pltpu.VMEM((2, _K), jnp.int32),
            pltpu.VMEM((_K, 16), jnp.float32),
            pltpu.VMEM((32, 16), jnp.float32),
            pltpu.VMEM_SHARED((_NPAD, 16), jnp.float32),
        ],
    )
    def k(dst_hbm, out_hbm, didx, ones, zbuf, acc):
        c = lax.axis_index("c")
        s = lax.axis_index("s")
        wid = s * 2 + c

        _zero_fill(zbuf, 32, 16)
        one = jnp.ones((16,), jnp.float32)

        @pl.loop(0, _K)
        def _(i):
            ones[i, pl.ds(0, 16)] = one

        @pl.loop(0, _RPW, step=32)
        def _(r):
            pltpu.sync_copy(zbuf, acc.at[pl.ds(s * _RPW + r, 32)])

        plsc.subcore_barrier()

        @pl.loop(0, _CH)
        def _(j):
            pltpu.sync_copy(dst_hbm.at[wid, j], didx.at[0])
            pltpu.sync_copy(ones, acc.at[didx.at[0]], add=True)

        plsc.subcore_barrier()
        pltpu.sync_copy(acc.at[pl.ds(s * _RPW, _RPW)],
                        out_hbm.at[c].at[pl.ds(s * _RPW, _RPW)])

    return k(dst_idx)


def _tc(fn, out_shape, *args):
    return pl.pallas_call(
        fn, out_shape=jax.ShapeDtypeStruct(out_shape, jnp.float32))(*args)


def _dinv_body(h_ref, o_ref):
    deg = h_ref[0, :, 0:1] + h_ref[1, :, 0:1] + 1.0
    o_ref[...] = jnp.broadcast_to(lax.rsqrt(deg), o_ref.shape)


def _tc1_body(s1_ref, d_ref, w1_ref, b1_ref, wg_ref, o_ref):
    x = (s1_ref[0] + s1_ref[1]) * (1.0 / _N)
    h = jnp.maximum(jnp.dot(x, w1_ref[...],
                            preferred_element_type=jnp.float32) + b1_ref[...], 0.0)
    o_ref[...] = d_ref[...] * jnp.dot(h, wg_ref[...],
                                      preferred_element_type=jnp.float32)


def _tc2_body(s2_ref, u1_ref, d_ref, b_ref, o_ref):
    agg = s2_ref[0] + s2_ref[1] + u1_ref[...]
    o_ref[...] = jnp.maximum(d_ref[...] * agg + b_ref[...], 0.0)


def _tc3_body(s3_ref, d_ref, w2_ref, b2_ref, wg_ref, o_ref):
    x = (s3_ref[0] + s3_ref[1]) * (1.0 / _N)
    h = jnp.dot(x, w2_ref[...], preferred_element_type=jnp.float32) + b2_ref[...]
    o_ref[...] = d_ref[:, 0:64] * jnp.dot(h, wg_ref[...],
                                          preferred_element_type=jnp.float32)


def _tc4_body(s4_ref, u2_ref, d_ref, b_ref, o_ref):
    agg = s4_ref[0] + s4_ref[1] + u2_ref[...]
    z = (d_ref[:, 0:64] * agg + b_ref[...])[: _N]
    m = jnp.max(z, axis=1, keepdims=True)
    e = jnp.exp(z - m)
    lse = m + jnp.log(jnp.sum(e, axis=1, keepdims=True))
    o_ref[...] = z - lse


def kernel(x_in, edge_index_in, ego_edge_index,
           W_ego1, b_ego1, W_gcn1, b_gcn1,
           W_ego2, b_ego2, W_gcn2, b_gcn2):
    ei = edge_index_in.astype(jnp.int32)
    ee = ego_edge_index.astype(jnp.int32)
    n_e = ei.shape[1]

    def prep(idx, fill):
        pad = jnp.full((_EPAD - n_e,), fill, jnp.int32)
        return jnp.concatenate([idx, pad]).reshape(_NW, _CH, _K)

    ego_dst = prep(ee[0], _N)
    ego_src = prep(ee[1], 0)
    g_src = prep(ei[0], 0)
    g_dst = prep(ei[1], _N)

    hist = _hist_sc(g_dst)
    S1 = _segsum_sc(ego_src, ego_dst, x_in, 128)

    dinvb = _tc(_dinv_body, (_NPAD, 128), hist)
    u1 = _tc(_tc1_body, (_NPAD, 128), S1, dinvb,
             W_ego1, b_ego1.reshape(1, -1), W_gcn1)

    S2 = _segsum_sc(g_src, g_dst, u1, 128)
    x2 = _tc(_tc2_body, (_NPAD, 128), S2, u1, dinvb, b_gcn1.reshape(1, -1))

    S3 = _segsum_sc(ego_src, ego_dst, x2, 128)
    u2 = _tc(_tc3_body, (_NPAD, 64), S3, dinvb,
             W_ego2, b_ego2.reshape(1, -1), W_gcn2)

    S4 = _segsum_sc(g_src, g_dst, u2, 64)
    out = _tc(_tc4_body, (_N, 64), S4, u2, dinvb, b_gcn2.reshape(1, -1))
    return out


# speedup vs baseline: 23.0617x; 4.4227x over previous
import functools

import jax
import jax.numpy as jnp
from jax import lax
from jax.experimental import pallas as pl
from jax.experimental.pallas import tpu as pltpu
from jax.experimental.pallas import tpu_sc as plsc

_N = 10000
_NPAD = 10240
_NW = 32
_K = 128
_CH = 80
_R = 2
_RI = 4
_EPAD = _NW * _K * _CH
_RPW = _NPAD // 16

_sc_mesh = plsc.VectorSubcoreMesh(core_axis_name="c", subcore_axis_name="s")


def _zero_fill(buf, nrows, ncols):
    zero = jnp.zeros((16,), jnp.float32)

    @pl.loop(0, nrows)
    def _(i):
        @pl.loop(0, ncols, step=16)
        def _(j):
            buf[i, pl.ds(j, 16)] = zero


@functools.partial(jax.jit, static_argnums=(3,))
def _segsum_sc(src_idx, dst_idx, table, F):
    @functools.partial(
        pl.kernel,
        out_type=jax.ShapeDtypeStruct((2, _NPAD, F), jnp.float32),
        mesh=_sc_mesh,
        compiler_params=pltpu.CompilerParams(use_tc_tiling_on_sc=False),
        scratch_types=[
            pltpu.VMEM((_RI, _K), jnp.int32),
            pltpu.VMEM((_RI, _K), jnp.int32),
            pltpu.VMEM((_R, _K, F), jnp.float32),
            pltpu.VMEM((16, F), jnp.float32),
            pltpu.VMEM_SHARED((_NPAD, F), jnp.float32),
        ] + [pltpu.SemaphoreType.DMA] * (_R + _RI),
    )
    def k(src_hbm, dst_hbm, tab_hbm, out_hbm, sidx, didx, rows, zbuf, acc, *sems):
        gsem = sems[:_R]
        isem = sems[_R:]
        c = lax.axis_index("c")
        s = lax.axis_index("s")
        wid = s * 2 + c

        def issue_idx(jj, q):
            pltpu.async_copy(src_hbm.at[wid, jj], sidx.at[q], isem[q])
            pltpu.async_copy(dst_hbm.at[wid, jj], didx.at[q], isem[q])

        def wait_idx(q):
            pltpu.make_async_copy(src_hbm.at[wid, 0], sidx.at[q], isem[q]).wait()
            pltpu.make_async_copy(dst_hbm.at[wid, 0], didx.at[q], isem[q]).wait()

        def issue_gather(jj, q, b):
            pltpu.async_copy(tab_hbm.at[sidx.at[q]], rows.at[b], gsem[b])

        for q in range(_RI):
            issue_idx(q, q)
        for b in range(_R):
            wait_idx(b)
            issue_gather(b, b, b)

        _zero_fill(zbuf, 16, F)

        @pl.loop(0, _RPW, step=16)
        def _(r):
            pltpu.sync_copy(zbuf, acc.at[pl.ds(s * _RPW + r, 16)])

        plsc.subcore_barrier()

        @pl.loop(0, _CH, step=_RI)
        def _(j):
            for b in range(_RI):
                jj = j + b
                buf = b % _R
                pltpu.make_async_copy(
                    tab_hbm.at[sidx.at[b]], rows.at[buf], gsem[buf]).wait()
                pltpu.sync_copy(rows.at[buf], acc.at[didx.at[b]], add=True)

                @pl.when(jj + _RI < _CH)
                def _():
                    issue_idx(jj + _RI, b)

                @pl.when(jj + _R < _CH)
                def _():
                    wait_idx((b + _R) % _RI)
                    issue_gather(jj + _R, (b + _R) % _RI, buf)

        plsc.subcore_barrier()
        pltpu.sync_copy(acc.at[pl.ds(s * _RPW, _RPW)],
                        out_hbm.at[c].at[pl.ds(s * _RPW, _RPW)])

    return k(src_idx, dst_idx, table)


@jax.jit
def _hist_sc(dst_idx):
    @functools.partial(
        pl.kernel,
        out_type=jax.ShapeDtypeStruct((2, _NPAD, 16), jnp.float32),
        mesh=_sc_mesh,
        compiler_params=pltpu.CompilerParams(use_tc_tiling_on_sc=False),
        scratch_types=[
            pltpu.VMEM((_CH, _K), jnp.int32),
            pltpu.VMEM((_K, 16), jnp.float32),
            pltpu.VMEM((32, 16), jnp.float32),
            pltpu.VMEM_SHARED((_NPAD, 16), jnp.float32),
        ],
    )
    def k(dst_hbm, out_hbm, didx, ones, zbuf, acc):
        c = lax.axis_index("c")
        s = lax.axis_index("s")
        wid = s * 2 + c

        pltpu.sync_copy(dst_hbm.at[wid], didx)
        _zero_fill(zbuf, 32, 16)
        one = jnp.ones((16,), jnp.float32)

        @pl.loop(0, _K)
        def _(i):
            ones[i, pl.ds(0, 16)] = one

        @pl.loop(0, _RPW, step=32)
        def _(r):
            pltpu.sync_copy(zbuf, acc.at[pl.ds(s * _RPW + r, 32)])

        plsc.subcore_barrier()

        @pl.loop(0, _CH)
        def _(j):
            pltpu.sync_copy(ones, acc.at[didx.at[j]], add=True)

        plsc.subcore_barrier()
        pltpu.sync_copy(acc.at[pl.ds(s * _RPW, _RPW)],
                        out_hbm.at[c].at[pl.ds(s * _RPW, _RPW)])

    return k(dst_idx)


def _tc(fn, out_shape, *args):
    return pl.pallas_call(
        fn, out_shape=jax.ShapeDtypeStruct(out_shape, jnp.float32))(*args)


def _dinv_body(h_ref, o_ref):
    deg = h_ref[0, :, 0:1] + h_ref[1, :, 0:1] + 1.0
    o_ref[...] = jnp.broadcast_to(lax.rsqrt(deg), o_ref.shape)


def _tc1_body(s1_ref, d_ref, w1_ref, b1_ref, wg_ref, o_ref):
    x = (s1_ref[0] + s1_ref[1]) * (1.0 / _N)
    h = jnp.maximum(jnp.dot(x, w1_ref[...],
                            preferred_element_type=jnp.float32) + b1_ref[...], 0.0)
    o_ref[...] = d_ref[...] * jnp.dot(h, wg_ref[...],
                                      preferred_element_type=jnp.float32)


def _tc2_body(s2_ref, u1_ref, d_ref, b_ref, o_ref):
    agg = s2_ref[0] + s2_ref[1] + u1_ref[...]
    o_ref[...] = jnp.maximum(d_ref[...] * agg + b_ref[...], 0.0)


def _tc3_body(s3_ref, d_ref, w2_ref, b2_ref, wg_ref, o_ref):
    x = (s3_ref[0] + s3_ref[1]) * (1.0 / _N)
    h = jnp.dot(x, w2_ref[...], preferred_element_type=jnp.float32) + b2_ref[...]
    o_ref[...] = d_ref[:, 0:64] * jnp.dot(h, wg_ref[...],
                                          preferred_element_type=jnp.float32)


def _tc4_body(s4_ref, u2_ref, d_ref, b_ref, o_ref):
    agg = s4_ref[0] + s4_ref[1] + u2_ref[...]
    z = (d_ref[:, 0:64] * agg + b_ref[...])[: _N]
    m = jnp.max(z, axis=1, keepdims=True)
    e = jnp.exp(z - m)
    lse = m + jnp.log(jnp.sum(e, axis=1, keepdims=True))
    o_ref[...] = z - lse


def kernel(x_in, edge_index_in, ego_edge_index,
           W_ego1, b_ego1, W_gcn1, b_gcn1,
           W_ego2, b_ego2, W_gcn2, b_gcn2):
    ei = edge_index_in.astype(jnp.int32)
    ee = ego_edge_index.astype(jnp.int32)
    n_e = ei.shape[1]
    n_pad = _EPAD - n_e

    pad_src = jnp.arange(n_pad, dtype=jnp.int32) % _N
    pad_dst = _N + (jnp.arange(n_pad, dtype=jnp.int32) % (_NPAD - _N))

    def prep(idx, pad):
        flat = jnp.concatenate([idx, pad])
        return flat.reshape(_CH, _NW, _K).transpose(1, 0, 2)

    ego_dst = prep(ee[0], pad_dst)
    ego_src = prep(ee[1], pad_src)
    g_src = prep(ei[0], pad_src)
    g_dst = prep(ei[1], pad_dst)

    hist = _hist_sc(g_dst)
    S1 = _segsum_sc(ego_src, ego_dst, x_in, 128)

    dinvb = _tc(_dinv_body, (_NPAD, 128), hist)
    u1 = _tc(_tc1_body, (_NPAD, 128), S1, dinvb,
             W_ego1, b_ego1.reshape(1, -1), W_gcn1)

    S2 = _segsum_sc(g_src, g_dst, u1, 128)
    x2 = _tc(_tc2_body, (_NPAD, 128), S2, u1, dinvb, b_gcn1.reshape(1, -1))

    S3 = _segsum_sc(ego_src, ego_dst, x2, 128)
    u2 = _tc(_tc3_body, (_NPAD, 64), S3, dinvb,
             W_ego2, b_ego2.reshape(1, -1), W_gcn2)

    S4 = _segsum_sc(g_src, g_dst, u2, 64)
    out = _tc(_tc4_body, (_N, 64), S4, u2, dinvb, b_gcn2.reshape(1, -1))
    return out
